# Initial kernel scaffold; baseline (speedup 1.0000x reference)
import jax
import jax.numpy as jnp
from jax.experimental import pallas as pl


def kernel(x, edge_index, W1, b1, W2, b2, W3, b3, gamma, beta):
    raise NotImplementedError("write your pallas kernel here")


# speedup vs baseline: 8.7390x; 8.7390x over previous
import functools

import jax
import jax.numpy as jnp
from jax import lax
from jax.experimental import pallas as pl
from jax.experimental.pallas import tpu as pltpu
from jax.experimental.pallas import tpu_sc as plsc

N_NODES = 10000
N_EDGES = 320000
D = 128

NC = 2
NS = 16
NW = NC * NS

EDGES_PER_TILE = N_EDGES // NW
CHUNK = 80
NCHUNK = EDGES_PER_TILE // CHUNK
N_PAD = 10240
ROWS_PER_TILE = N_PAD // NS


def _sc_aggregate(x, edge_index, zeros):
    mesh = plsc.VectorSubcoreMesh(core_axis_name="c", subcore_axis_name="s",
                                  num_cores=NC, num_subcores=NS)

    @functools.partial(
        pl.kernel,
        out_type=jax.ShapeDtypeStruct((NC, N_PAD, D), jnp.float32),
        mesh=mesh,
        scratch_types=[
            pltpu.VMEM((CHUNK,), jnp.int32),
            pltpu.VMEM((CHUNK,), jnp.int32),
            pltpu.VMEM((CHUNK, D), jnp.float32),
            pltpu.VMEM_SHARED((N_PAD, D), jnp.float32),
            pltpu.SemaphoreType.DMA,
        ],
    )
    def agg_kernel(x_hbm, s_hbm, r_hbm, zeros_hbm, out_hbm, si_v, ri_v, rows_v,
                   acc_sh, sem):
        cid = lax.axis_index("c")
        sid = lax.axis_index("s")
        wid = cid * NS + sid

        rbase = sid * ROWS_PER_TILE
        pltpu.sync_copy(zeros_hbm.at[pl.ds(rbase, ROWS_PER_TILE)],
                        acc_sh.at[pl.ds(rbase, ROWS_PER_TILE)])
        plsc.subcore_barrier()

        ebase = wid * EDGES_PER_TILE

        @pl.loop(0, NCHUNK)
        def _(i):
            off = ebase + i * CHUNK
            pltpu.sync_copy(s_hbm.at[pl.ds(off, CHUNK)], si_v)
            pltpu.sync_copy(r_hbm.at[pl.ds(off, CHUNK)], ri_v)
            pltpu.async_copy(x_hbm.at[si_v], rows_v, sem).wait()
            pltpu.sync_copy(rows_v, acc_sh.at[ri_v], add=True)
            pltpu.async_copy(x_hbm.at[ri_v], rows_v, sem).wait()
            pltpu.sync_copy(rows_v, acc_sh.at[si_v], add=True)

        plsc.subcore_barrier()
        pltpu.sync_copy(acc_sh.at[pl.ds(rbase, ROWS_PER_TILE)],
                        out_hbm.at[cid, pl.ds(rbase, ROWS_PER_TILE)])

    return agg_kernel(x, edge_index[0], edge_index[1], zeros)


BLK = 1000


def _gelu_exact(v):
    return 0.5 * v * (1.0 + lax.erf(v * 0.7071067811865476))


def _mlp_body(p_ref, w1_ref, b1_ref, w2_ref, b2_ref, w3_ref, b3_ref,
              g_ref, bt_ref, o_ref):
    agg = p_ref[0] + p_ref[1]
    h = jnp.dot(agg, w1_ref[:], preferred_element_type=jnp.float32) + b1_ref[:]
    h = _gelu_exact(h)
    h = jnp.dot(h, w2_ref[:], preferred_element_type=jnp.float32) + b2_ref[:]
    h = _gelu_exact(h)
    o = jnp.dot(h, w3_ref[:], preferred_element_type=jnp.float32) + b3_ref[:]
    mu = jnp.mean(o, axis=-1, keepdims=True)
    var = jnp.mean((o - mu) ** 2, axis=-1, keepdims=True)
    o_ref[:] = (o - mu) / jnp.sqrt(var + 1e-5) * g_ref[:] + bt_ref[:]


def _tc_mlp(parts, W1, b1, W2, b2, W3, b3, gamma, beta):
    vec = pl.BlockSpec((1, D), lambda i: (0, 0))
    mat = pl.BlockSpec((D, D), lambda i: (0, 0))
    return pl.pallas_call(
        _mlp_body,
        grid=(N_NODES // BLK,),
        in_specs=[pl.BlockSpec((NC, BLK, D), lambda i: (0, i, 0)),
                  mat, vec, mat, vec, mat, vec, vec, vec],
        out_specs=pl.BlockSpec((BLK, D), lambda i: (i, 0)),
        out_shape=jax.ShapeDtypeStruct((N_NODES, D), jnp.float32),
    )(parts, W1, b1.reshape(1, D), W2, b2.reshape(1, D),
      W3, b3.reshape(1, D), gamma.reshape(1, D), beta.reshape(1, D))


def kernel(x, edge_index, W1, b1, W2, b2, W3, b3, gamma, beta):
    ei = edge_index.astype(jnp.int32)
    zeros = jnp.zeros((N_PAD, D), jnp.float32)
    parts = _sc_aggregate(x, ei, zeros)
    return _tc_mlp(parts, W1, b1, W2, b2, W3, b3, gamma, beta)


# speedup vs baseline: 13.5591x; 1.5516x over previous
import functools

import jax
import jax.numpy as jnp
from jax import lax
from jax.experimental import pallas as pl
from jax.experimental.pallas import tpu as pltpu
from jax.experimental.pallas import tpu_sc as plsc

N_NODES = 10000
N_EDGES = 320000
D = 128

NC = 2
NS = 16
NW = NC * NS

EDGES_PER_TILE = N_EDGES // NW
CHUNK = 80
NCHUNK = EDGES_PER_TILE // CHUNK
N_PAD = 10112
ROWS_PER_TILE = N_PAD // NS


def _sc_aggregate(x, edge_index, zeros):
    mesh = plsc.VectorSubcoreMesh(core_axis_name="c", subcore_axis_name="s",
                                  num_cores=NC, num_subcores=NS)

    @functools.partial(
        pl.kernel,
        out_type=jax.ShapeDtypeStruct((NC, N_PAD, D), jnp.float32),
        mesh=mesh,
        scratch_types=[
            pltpu.VMEM((CHUNK,), jnp.int32),
            pltpu.VMEM((CHUNK,), jnp.int32),
            pltpu.VMEM((CHUNK,), jnp.int32),
            pltpu.VMEM((CHUNK,), jnp.int32),
            pltpu.VMEM((CHUNK, D), jnp.float32),
            pltpu.VMEM((CHUNK, D), jnp.float32),
            pltpu.VMEM((CHUNK, D), jnp.float32),
            pltpu.VMEM((CHUNK, D), jnp.float32),
            pltpu.VMEM_SHARED((N_PAD, D), jnp.float32),
        ] + [pltpu.SemaphoreType.DMA] * 8,
    )
    def agg_kernel(x_hbm, s_hbm, r_hbm, zeros_hbm, out_hbm,
                   si0, si1, ri0, ri1, ra0, ra1, rb0, rb1, acc_sh,
                   ga0, ga1, gb0, gb1, sa0, sa1, sb0, sb1):
        cid = lax.axis_index("c")
        sid = lax.axis_index("s")
        wid = cid * NS + sid
        si, ri = (si0, si1), (ri0, ri1)
        rows_a, rows_b = (ra0, ra1), (rb0, rb1)
        gsem_a, gsem_b = (ga0, ga1), (gb0, gb1)
        isem, rsem = (sa0, sa1), (sb0, sb1)
        cbase = wid * NCHUNK

        rbase = sid * ROWS_PER_TILE
        pltpu.sync_copy(zeros_hbm.at[pl.ds(rbase, ROWS_PER_TILE)],
                        acc_sh.at[pl.ds(rbase, ROWS_PER_TILE)])
        plsc.subcore_barrier()

        def group(base_chunk, n_chunks):
            idescs = []
            for j in range(n_chunks):
                off = base_chunk * CHUNK + j * CHUNK
                idescs.append(pltpu.async_copy(
                    s_hbm.at[pl.ds(off, CHUNK)], si[j], isem[j]))
                idescs.append(pltpu.async_copy(
                    r_hbm.at[pl.ds(off, CHUNK)], ri[j], rsem[j]))
            gdescs = []
            for j in range(n_chunks):
                idescs[2 * j].wait()
                idescs[2 * j + 1].wait()
                gdescs.append(pltpu.async_copy(
                    x_hbm.at[si[j]], rows_a[j], gsem_a[j]))
                gdescs.append(pltpu.async_copy(
                    x_hbm.at[ri[j]], rows_b[j], gsem_b[j]))
            for j in range(n_chunks):
                gdescs[2 * j].wait()
                pltpu.sync_copy(rows_a[j], acc_sh.at[ri[j]], add=True)
                gdescs[2 * j + 1].wait()
                pltpu.sync_copy(rows_b[j], acc_sh.at[si[j]], add=True)

        @pl.loop(0, NCHUNK // 2)
        def _(g):
            group(cbase + 2 * g, 2)

        group(cbase + NCHUNK - 1, 1)

        plsc.subcore_barrier()
        pltpu.sync_copy(acc_sh.at[pl.ds(rbase, ROWS_PER_TILE)],
                        out_hbm.at[cid, pl.ds(rbase, ROWS_PER_TILE)])

    return agg_kernel(x, edge_index[0], edge_index[1], zeros)


BLK = 1000


def _gelu_exact(v):
    return 0.5 * v * (1.0 + lax.erf(v * 0.7071067811865476))


def _mlp_body(p_ref, w1_ref, b1_ref, w2_ref, b2_ref, w3_ref, b3_ref,
              g_ref, bt_ref, o_ref):
    agg = p_ref[0] + p_ref[1]
    h = jnp.dot(agg, w1_ref[:], preferred_element_type=jnp.float32) + b1_ref[:]
    h = _gelu_exact(h)
    h = jnp.dot(h, w2_ref[:], preferred_element_type=jnp.float32) + b2_ref[:]
    h = _gelu_exact(h)
    o = jnp.dot(h, w3_ref[:], preferred_element_type=jnp.float32) + b3_ref[:]
    mu = jnp.mean(o, axis=-1, keepdims=True)
    var = jnp.mean((o - mu) ** 2, axis=-1, keepdims=True)
    o_ref[:] = (o - mu) / jnp.sqrt(var + 1e-5) * g_ref[:] + bt_ref[:]


def _tc_mlp(parts, W1, b1, W2, b2, W3, b3, gamma, beta):
    vec = pl.BlockSpec((1, D), lambda i: (0, 0))
    mat = pl.BlockSpec((D, D), lambda i: (0, 0))
    return pl.pallas_call(
        _mlp_body,
        grid=(N_NODES // BLK,),
        in_specs=[pl.BlockSpec((NC, BLK, D), lambda i: (0, i, 0)),
                  mat, vec, mat, vec, mat, vec, vec, vec],
        out_specs=pl.BlockSpec((BLK, D), lambda i: (i, 0)),
        out_shape=jax.ShapeDtypeStruct((N_NODES, D), jnp.float32),
    )(parts, W1, b1.reshape(1, D), W2, b2.reshape(1, D),
      W3, b3.reshape(1, D), gamma.reshape(1, D), beta.reshape(1, D))


def kernel(x, edge_index, W1, b1, W2, b2, W3, b3, gamma, beta):
    ei = edge_index.astype(jnp.int32)
    zeros = jnp.zeros((N_PAD, D), jnp.float32)
    parts = _sc_aggregate(x, ei, zeros)
    return _tc_mlp(parts, W1, b1, W2, b2, W3, b3, gamma, beta)


# speedup vs baseline: 13.9512x; 1.0289x over previous
import functools

import jax
import jax.numpy as jnp
from jax import lax
from jax.experimental import pallas as pl
from jax.experimental.pallas import tpu as pltpu
from jax.experimental.pallas import tpu_sc as plsc

N_NODES = 10000
N_EDGES = 320000
D = 128

NC = 2
NS = 16
NW = NC * NS

EDGES_PER_TILE = N_EDGES // NW
CHUNK = 80
NCHUNK = EDGES_PER_TILE // CHUNK
N_PAD = 10112
ROWS_PER_TILE = N_PAD // NS


def _sc_aggregate(x, edge_index, zeros):
    mesh = plsc.VectorSubcoreMesh(core_axis_name="c", subcore_axis_name="s",
                                  num_cores=NC, num_subcores=NS)

    @functools.partial(
        pl.kernel,
        out_type=jax.ShapeDtypeStruct((NC, N_PAD, D), jnp.float32),
        mesh=mesh,
        scratch_types=[
            pltpu.VMEM((CHUNK,), jnp.int32),
            pltpu.VMEM((CHUNK,), jnp.int32),
            pltpu.VMEM((CHUNK,), jnp.int32),
            pltpu.VMEM((CHUNK,), jnp.int32),
            pltpu.VMEM((CHUNK, D), jnp.float32),
            pltpu.VMEM((CHUNK, D), jnp.float32),
            pltpu.VMEM((CHUNK, D), jnp.float32),
            pltpu.VMEM((CHUNK, D), jnp.float32),
            pltpu.VMEM_SHARED((N_PAD, D), jnp.float32),
        ] + [pltpu.SemaphoreType.DMA] * 12,
    )
    def agg_kernel(x_hbm, s_hbm, r_hbm, zeros_hbm, out_hbm,
                   si0, si1, ri0, ri1, ra0, ra1, rb0, rb1, acc_sh,
                   ga0, ga1, gb0, gb1, ia0, ia1, ib0, ib1,
                   sa0, sa1, sb0, sb1):
        cid = lax.axis_index("c")
        sid = lax.axis_index("s")
        wid = cid * NS + sid
        si, ri = (si0, si1), (ri0, ri1)
        rows_a, rows_b = (ra0, ra1), (rb0, rb1)
        gsem_a, gsem_b = (ga0, ga1), (gb0, gb1)
        isem, rsem = (ia0, ia1), (ib0, ib1)
        ssem_a, ssem_b = (sa0, sa1), (sb0, sb1)
        cbase = wid * NCHUNK

        rbase = sid * ROWS_PER_TILE
        pltpu.sync_copy(zeros_hbm.at[pl.ds(rbase, ROWS_PER_TILE)],
                        acc_sh.at[pl.ds(rbase, ROWS_PER_TILE)])
        plsc.subcore_barrier()

        def group(base_chunk, n_chunks):
            idescs = []
            for j in range(n_chunks):
                off = base_chunk * CHUNK + j * CHUNK
                idescs.append(pltpu.async_copy(
                    s_hbm.at[pl.ds(off, CHUNK)], si[j], isem[j]))
                idescs.append(pltpu.async_copy(
                    r_hbm.at[pl.ds(off, CHUNK)], ri[j], rsem[j]))
            gdescs = []
            for j in range(n_chunks):
                idescs[2 * j].wait()
                idescs[2 * j + 1].wait()
                gdescs.append(pltpu.async_copy(
                    x_hbm.at[si[j]], rows_a[j], gsem_a[j]))
                gdescs.append(pltpu.async_copy(
                    x_hbm.at[ri[j]], rows_b[j], gsem_b[j]))
            sdescs = []
            for j in range(n_chunks):
                gdescs[2 * j].wait()
                sdescs.append(pltpu.async_copy(
                    rows_a[j], acc_sh.at[ri[j]], ssem_a[j], add=True))
                gdescs[2 * j + 1].wait()
                sdescs.append(pltpu.async_copy(
                    rows_b[j], acc_sh.at[si[j]], ssem_b[j], add=True))
            for d in sdescs:
                d.wait()

        @pl.loop(0, NCHUNK // 2)
        def _(g):
            group(cbase + 2 * g, 2)

        group(cbase + NCHUNK - 1, 1)

        plsc.subcore_barrier()
        pltpu.sync_copy(acc_sh.at[pl.ds(rbase, ROWS_PER_TILE)],
                        out_hbm.at[cid, pl.ds(rbase, ROWS_PER_TILE)])

    return agg_kernel(x, edge_index[0], edge_index[1], zeros)


BLK = 1000


def _gelu_exact(v):
    return 0.5 * v * (1.0 + lax.erf(v * 0.7071067811865476))


def _mlp_body(p_ref, w1_ref, b1_ref, w2_ref, b2_ref, w3_ref, b3_ref,
              g_ref, bt_ref, o_ref):
    agg = p_ref[0] + p_ref[1]
    h = jnp.dot(agg, w1_ref[:], preferred_element_type=jnp.float32) + b1_ref[:]
    h = _gelu_exact(h)
    h = jnp.dot(h, w2_ref[:], preferred_element_type=jnp.float32) + b2_ref[:]
    h = _gelu_exact(h)
    o = jnp.dot(h, w3_ref[:], preferred_element_type=jnp.float32) + b3_ref[:]
    mu = jnp.mean(o, axis=-1, keepdims=True)
    var = jnp.mean((o - mu) ** 2, axis=-1, keepdims=True)
    o_ref[:] = (o - mu) / jnp.sqrt(var + 1e-5) * g_ref[:] + bt_ref[:]


def _tc_mlp(parts, W1, b1, W2, b2, W3, b3, gamma, beta):
    vec = pl.BlockSpec((1, D), lambda i: (0, 0))
    mat = pl.BlockSpec((D, D), lambda i: (0, 0))
    return pl.pallas_call(
        _mlp_body,
        grid=(N_NODES // BLK,),
        in_specs=[pl.BlockSpec((NC, BLK, D), lambda i: (0, i, 0)),
                  mat, vec, mat, vec, mat, vec, vec, vec],
        out_specs=pl.BlockSpec((BLK, D), lambda i: (i, 0)),
        out_shape=jax.ShapeDtypeStruct((N_NODES, D), jnp.float32),
    )(parts, W1, b1.reshape(1, D), W2, b2.reshape(1, D),
      W3, b3.reshape(1, D), gamma.reshape(1, D), beta.reshape(1, D))


def kernel(x, edge_index, W1, b1, W2, b2, W3, b3, gamma, beta):
    ei = edge_index.astype(jnp.int32)
    zeros = jnp.zeros((N_PAD, D), jnp.float32)
    parts = _sc_aggregate(x, ei, zeros)
    return _tc_mlp(parts, W1, b1, W2, b2, W3, b3, gamma, beta)
